# Initial kernel scaffold; baseline (speedup 1.0000x reference)
import jax
import jax.numpy as jnp
from jax.experimental import pallas as pl


def kernel(x, edge_index, edge_attr, batch, W1_rel, b1, W1_root, W2_rel, b2, W2_root, W_lin, b_lin):
    raise NotImplementedError("write your pallas kernel here")


# speedup vs baseline: 5.1782x; 5.1782x over previous
import functools

import jax
import jax.numpy as jnp
from jax import lax
from jax.experimental import pallas as pl
from jax.experimental.pallas import tpu as pltpu
from jax.experimental.pallas import tpu_sc as plsc


@functools.cache
def _make_edge_agg(N: int, D: int, E: int):
    info = plsc.get_sparse_core_info()
    NC, NS = info.num_cores, info.num_subcores
    NT = NC * NS
    assert E % NT == 0
    e_per = E // NT
    CH = 80
    assert e_per % CH == 0 and CH % 8 == 0
    n_ch = e_per // CH
    ZR = 80
    assert N % ZR == 0
    n_zc = N // ZR
    zc_per = -(-n_zc // NS)

    mesh = plsc.VectorSubcoreMesh(core_axis_name="c", subcore_axis_name="s")

    @functools.partial(
        pl.kernel,
        out_type=jax.ShapeDtypeStruct((NC, N, D), jnp.float32),
        mesh=mesh,
        scratch_types=[
            pltpu.VMEM((CH,), jnp.int32),
            pltpu.VMEM((CH,), jnp.int32),
            pltpu.VMEM((CH, D), jnp.float32),
            pltpu.VMEM((80, D), jnp.float32),
            pltpu.VMEM_SHARED((N, D), jnp.float32),
            pltpu.SemaphoreType.DMA,
        ],
    )
    def edge_agg(src_hbm, dst_hbm, x_hbm, out_hbm, src_v, dst_v, rows_v, zb_v, acc_sh, sem):
        cid = lax.axis_index("c")
        sid = lax.axis_index("s")
        wid = sid * NC + cid

        zero16 = jnp.zeros((16,), jnp.float32)

        @pl.loop(0, ZR)
        def _zero_zb(i):
            for j in range(D // 16):
                zb_v[i, pl.ds(j * 16, 16)] = zero16

        @pl.loop(0, zc_per)
        def _zero_acc(k):
            c = sid + k * NS

            @pl.when(c < n_zc)
            def _():
                pltpu.sync_copy(zb_v, acc_sh.at[pl.ds(c * ZR, ZR)])

        plsc.subcore_barrier()

        e0 = wid * e_per

        @pl.loop(0, n_ch)
        def _edges(j):
            base = e0 + j * CH
            pltpu.sync_copy(src_hbm.at[pl.ds(base, CH)], src_v)
            pltpu.sync_copy(dst_hbm.at[pl.ds(base, CH)], dst_v)
            pltpu.async_copy(x_hbm.at[src_v], rows_v, sem).wait()
            pltpu.sync_copy(rows_v, acc_sh.at[dst_v], add=True)

        plsc.subcore_barrier()

        @pl.loop(0, zc_per)
        def _writeout(k):
            c = sid + k * NS

            @pl.when(c < n_zc)
            def _():
                r = c * ZR
                pltpu.sync_copy(acc_sh.at[pl.ds(r, ZR)], zb_v)
                pltpu.sync_copy(zb_v, out_hbm.at[cid, pl.ds(r, ZR)])

    return edge_agg


def _tc_layer1(p, x, w_rel, b, w_root, bn: int):
    N, D = x.shape
    H = w_rel.shape[1]
    grid = N // bn

    def body(p_ref, x_ref, wrel_ref, wroot_ref, b_ref, o_ref):
        agg = p_ref[0] + p_ref[1]
        acc = jnp.dot(agg, wrel_ref[...], preferred_element_type=jnp.float32)
        acc = acc + jnp.dot(x_ref[...], wroot_ref[...], preferred_element_type=jnp.float32)
        o_ref[...] = jnp.maximum(acc + b_ref[...], 0.0)

    return pl.pallas_call(
        body,
        grid=(grid,),
        in_specs=[
            pl.BlockSpec((2, bn, D), lambda i: (0, i, 0)),
            pl.BlockSpec((bn, D), lambda i: (i, 0)),
            pl.BlockSpec((D, H), lambda i: (0, 0)),
            pl.BlockSpec((D, H), lambda i: (0, 0)),
            pl.BlockSpec((1, H), lambda i: (0, 0)),
        ],
        out_specs=pl.BlockSpec((bn, H), lambda i: (i, 0)),
        out_shape=jax.ShapeDtypeStruct((N, H), jnp.float32),
    )(p, x, w_rel, w_root, b.reshape(1, H))


def _tc_layer2_pool(q, h1, w_rel, b, w_root, batch2d, w_lin, b_lin, G: int, bn: int):
    N, H = h1.shape
    C = w_lin.shape[1]
    grid = N // bn

    def body(q_ref, h_ref, wrel_ref, wroot_ref, b_ref, batch_ref, wlin_ref, blin_ref,
             o_ref, sums_ref, cnts_ref):
        i = pl.program_id(0)

        @pl.when(i == 0)
        def _init():
            sums_ref[...] = jnp.zeros_like(sums_ref)
            cnts_ref[...] = jnp.zeros_like(cnts_ref)

        agg = q_ref[0] + q_ref[1]
        h2 = jnp.dot(agg, wrel_ref[...], preferred_element_type=jnp.float32)
        h2 = h2 + jnp.dot(h_ref[...], wroot_ref[...], preferred_element_type=jnp.float32)
        h2 = h2 + b_ref[...]

        gids = lax.broadcasted_iota(jnp.int32, (bn, G), 1)
        oh = (batch_ref[...] == gids).astype(jnp.float32)
        sums_ref[...] += lax.dot_general(
            oh, h2, (((0,), (0,)), ((), ())), preferred_element_type=jnp.float32)
        cnts_ref[...] += jnp.sum(oh, axis=0, keepdims=True)

        @pl.when(i == grid - 1)
        def _final():
            pooled = sums_ref[...] / jnp.maximum(cnts_ref[...].reshape(G, 1), 1.0)
            o_ref[...] = (
                jnp.dot(pooled, wlin_ref[...], preferred_element_type=jnp.float32)
                + blin_ref[...]
            )

    return pl.pallas_call(
        body,
        grid=(grid,),
        in_specs=[
            pl.BlockSpec((2, bn, H), lambda i: (0, i, 0)),
            pl.BlockSpec((bn, H), lambda i: (i, 0)),
            pl.BlockSpec((H, H), lambda i: (0, 0)),
            pl.BlockSpec((H, H), lambda i: (0, 0)),
            pl.BlockSpec((1, H), lambda i: (0, 0)),
            pl.BlockSpec((bn, 1), lambda i: (i, 0)),
            pl.BlockSpec((H, C), lambda i: (0, 0)),
            pl.BlockSpec((1, C), lambda i: (0, 0)),
        ],
        out_specs=pl.BlockSpec((G, C), lambda i: (0, 0)),
        out_shape=jax.ShapeDtypeStruct((G, C), jnp.float32),
        scratch_shapes=[
            pltpu.VMEM((G, H), jnp.float32),
            pltpu.VMEM((1, G), jnp.float32),
        ],
    )(q, h1, w_rel, w_root, b.reshape(1, H), batch2d, w_lin, b_lin.reshape(1, C))


def kernel(x, edge_index, edge_attr, batch, W1_rel, b1, W1_root,
           W2_rel, b2, W2_root, W_lin, b_lin):
    del edge_attr
    N, D = x.shape
    E = edge_index.shape[1]
    H = W1_rel.shape[1]
    G = 32
    C = W_lin.shape[1]
    BN = 400

    src = edge_index[0]
    dst = edge_index[1]
    batch2d = batch.reshape(N, 1)

    edge_agg = _make_edge_agg(N, D, E)
    p = edge_agg(src, dst, x)
    h1 = _tc_layer1(p, x, W1_rel, b1, W1_root, BN)
    q = edge_agg(src, dst, h1)
    return _tc_layer2_pool(q, h1, W2_rel, b2, W2_root, batch2d, W_lin, b_lin, G, BN)
